# Initial kernel scaffold; baseline (speedup 1.0000x reference)
import functools

import jax
import jax.numpy as jnp
from jax import lax
from jax.experimental import pallas as pl
from jax.experimental.pallas import tpu as pltpu
from jax.experimental.pallas import tpu_sc as plsc

B = 16384
OUT_D = 128
NAME_ROWS = 10
TYPE_ROWS = 8
EMB_D = 5
N_COMB = NAME_ROWS * TYPE_ROWS

_NC = 2
_NS = 16
_L = 16
_NW = _NC * _NS
_BPW = B // _NW
_CHUNK = 128
_NCHUNK = _BPW // _CHUNK


def _table_body(name_ref, type_ref, w_ref, b_ref, out_ref):
    name = name_ref[...]
    typ = type_ref[...]
    w = w_ref[...]
    dn = (((1,), (1,)), ((), ()))
    a = lax.dot_general(name, w[:, :EMB_D], dn,
                        preferred_element_type=jnp.float32)
    c = lax.dot_general(typ, w[:, EMB_D:], dn,
                        preferred_element_type=jnp.float32)
    s = a[:, None, :] + c[None, :, :] + b_ref[0][None, None, :]
    out_ref[...] = jax.nn.sigmoid(s)


def _build_table(name_table, type_table, W, b):
    out = pl.pallas_call(
        _table_body,
        out_shape=jax.ShapeDtypeStruct((NAME_ROWS, TYPE_ROWS, OUT_D),
                                       jnp.float32),
    )(name_table, type_table, W, b.reshape(1, OUT_D))
    return out.reshape(N_COMB, OUT_D)


def _gather_body(f0_hbm, f1_hbm, table_hbm, out_hbm,
                 f0_v, f1_v, idx_v, rows_v, sem):
    wid = lax.axis_index("s") * _NC + lax.axis_index("c")
    base = wid * _BPW
    pltpu.sync_copy(f0_hbm.at[pl.ds(base, _BPW)], f0_v)
    pltpu.sync_copy(f1_hbm.at[pl.ds(base, _BPW)], f1_v)
    for c in range(_BPW // _L):
        r, off = divmod(c * _L, _CHUNK)
        s = pl.ds(c * _L, _L)
        idx_v[r, pl.ds(off, _L)] = f0_v[s] * TYPE_ROWS + f1_v[s]
    copies = [
        pltpu.async_copy(table_hbm.at[idx_v.at[k]],
                         rows_v.at[pl.ds(k * _CHUNK, _CHUNK)], sem)
        for k in range(_NCHUNK)
    ]
    for cp in copies:
        cp.wait()
    pltpu.sync_copy(rows_v, out_hbm.at[pl.ds(base, _BPW)])


_sc_gather = functools.partial(
    pl.kernel,
    out_type=jax.ShapeDtypeStruct((B, OUT_D), jnp.float32),
    mesh=plsc.VectorSubcoreMesh(core_axis_name="c", subcore_axis_name="s"),
    scratch_types=[
        pltpu.VMEM((_BPW,), jnp.int32),
        pltpu.VMEM((_BPW,), jnp.int32),
        pltpu.VMEM((_NCHUNK, _CHUNK), jnp.int32),
        pltpu.VMEM((_BPW, OUT_D), jnp.float32),
        pltpu.SemaphoreType.DMA,
    ],
)(_gather_body)


@jax.jit
def kernel(features, name_table, type_table, W, b):
    feats = features.astype(jnp.int32)
    table = _build_table(name_table, type_table, W, b)
    return _sc_gather(feats[0], feats[1], table)


# speedup vs baseline: 3.1817x; 3.1817x over previous
import functools

import jax
import jax.numpy as jnp
from jax import lax
from jax.experimental import pallas as pl
from jax.experimental.pallas import tpu as pltpu
from jax.experimental.pallas import tpu_sc as plsc

B = 16384
OUT_D = 128
NAME_ROWS = 10
TYPE_ROWS = 8
EMB_D = 5
N_COMB = NAME_ROWS * TYPE_ROWS

_NC = 2
_NS = 16
_L = 16
_NW = _NC * _NS
_BPW = B // _NW
_CHUNK = 128
_NCHUNK = _BPW // _CHUNK


def _table_body(name_ref, type_ref, w_ref, b_ref, out_ref):
    name = name_ref[...]
    typ = type_ref[...]
    w = w_ref[...]
    dn = (((1,), (1,)), ((), ()))
    a = lax.dot_general(name, w[:, :EMB_D], dn,
                        preferred_element_type=jnp.float32)
    c = lax.dot_general(typ, w[:, EMB_D:], dn,
                        preferred_element_type=jnp.float32)
    s = a[:, None, :] + c[None, :, :] + b_ref[0][None, None, :]
    out_ref[...] = jax.nn.sigmoid(s)


def _build_table(name_table, type_table, W, b):
    out = pl.pallas_call(
        _table_body,
        out_shape=jax.ShapeDtypeStruct((NAME_ROWS, TYPE_ROWS, OUT_D),
                                       jnp.float32),
    )(name_table, type_table, W, b.reshape(1, OUT_D))
    return out.reshape(N_COMB, OUT_D)


def _gather_body(f0_hbm, f1_hbm, table_hbm, out_hbm,
                 f0_v, f1_v, idx_v, rows_v, sem):
    wid = lax.axis_index("s") * _NC + lax.axis_index("c")
    base = wid * _BPW
    pltpu.sync_copy(f0_hbm.at[pl.ds(base, _BPW)], f0_v)
    pltpu.sync_copy(f1_hbm.at[pl.ds(base, _BPW)], f1_v)
    for c in range(_BPW // _L):
        r, off = divmod(c * _L, _CHUNK)
        s = pl.ds(c * _L, _L)
        idx_v[r, pl.ds(off, _L)] = f0_v[s] * TYPE_ROWS + f1_v[s]
    copies = [
        pltpu.async_copy(table_hbm.at[idx_v.at[k]],
                         rows_v.at[pl.ds(k * _CHUNK, _CHUNK)], sem)
        for k in range(_NCHUNK)
    ]
    for cp in copies:
        cp.wait()
    pltpu.sync_copy(rows_v, out_hbm.at[pl.ds(base, _BPW)])


def _sc_gather():
    return functools.partial(
        pl.kernel,
        out_type=jax.ShapeDtypeStruct((B, OUT_D), jnp.float32),
        mesh=plsc.VectorSubcoreMesh(core_axis_name="c", subcore_axis_name="s"),
        scratch_types=[
            pltpu.VMEM((_BPW,), jnp.int32),
            pltpu.VMEM((_BPW,), jnp.int32),
            pltpu.VMEM((_NCHUNK, _CHUNK), jnp.int32),
            pltpu.VMEM((_BPW, OUT_D), jnp.float32),
            pltpu.SemaphoreType.DMA,
        ],
    )(_gather_body)


@jax.jit
def kernel(features, name_table, type_table, W, b):
    feats = features.astype(jnp.int32)
    table = _build_table(name_table, type_table, W, b)
    return _sc_gather()(feats[0], feats[1], table)
